# baseline (speedup 1.0000x reference)
import functools

import jax
import jax.numpy as jnp
from jax import lax
from jax.experimental import pallas as pl
from jax.experimental.pallas import tpu as pltpu
from jax.experimental.pallas import tpu_sc as plsc

_SCALE = 30.0
_R2 = 0.7071067811865476
_LOG2E = 1.4426950408889634
_A = _SCALE * _LOG2E


def _sc_gather(yh_flat, y, num_class):
    b = y.shape[0]
    nw = 32
    per_w = b // nw
    mesh = plsc.VectorSubcoreMesh(core_axis_name="c", subcore_axis_name="s")

    @functools.partial(
        pl.kernel,
        mesh=mesh,
        out_type=jax.ShapeDtypeStruct((b,), jnp.float32),
        scratch_types=[
            pltpu.VMEM((per_w,), jnp.int32),
            pltpu.VMEM((per_w,), jnp.int32),
            pltpu.VMEM((per_w,), jnp.float32),
            pltpu.SemaphoreType.DMA,
        ],
    )
    def k(yh_hbm, y_hbm, out_hbm, yv, idxv, cv, sem):
        wid = lax.axis_index("s") * 2 + lax.axis_index("c")
        base = wid * per_w
        pltpu.sync_copy(y_hbm.at[pl.ds(base, per_w)], yv)
        for j in range(per_w // 16):
            rows = lax.iota(jnp.int32, 16) + (base + j * 16)
            idxv[pl.ds(j * 16, 16)] = (
                rows * num_class + yv[pl.ds(j * 16, 16)]
            )
        for t in range(per_w // 128):
            pltpu.async_copy(
                yh_hbm.at[idxv.at[pl.ds(t * 128, 128)]],
                cv.at[pl.ds(t * 128, 128)],
                sem,
            ).wait()
        pltpu.sync_copy(cv, out_hbm.at[pl.ds(base, per_w)])

    return k(yh_flat, y)


def _psi(c):
    c = jnp.clip(c, -1.0, 1.0)
    c2 = c * c
    cos4 = 8.0 * c2 * c2 - 8.0 * c2 + 1.0
    k = (
        (c <= _R2).astype(jnp.int32)
        + (c <= 0.0).astype(jnp.int32)
        + (c <= -_R2).astype(jnp.int32)
    )
    co = jnp.where((k & 1) == 1, -1.0, 1.0)
    return co * cos4 - 2.0 * k.astype(jnp.float32)


def _sub_loss(yh, c):
    psi = _psi(c)
    s0 = jnp.sum(jnp.exp2(yh * _A), axis=1, keepdims=True)
    s = s0 - jnp.exp2(c * _A) + jnp.exp2(psi * _A)
    lse = jnp.log(s)
    return jnp.sum(lse - _SCALE * psi)


def _body(a_ref, b_ref, c_ref, d_ref, ca_ref, cb_ref, cc_ref, cd_ref, out_ref):
    i = pl.program_id(0)
    nsteps = pl.num_programs(0)

    part = (
        _sub_loss(a_ref[...], ca_ref[...])
        + _sub_loss(b_ref[...], cb_ref[...])
        + _sub_loss(c_ref[...], cc_ref[...])
        + _sub_loss(d_ref[...], cd_ref[...])
    )

    @pl.when(i == 0)
    def _init():
        out_ref[0, 0] = 0.0

    out_ref[0, 0] += part

    @pl.when(i == nsteps - 1)
    def _final():
        out_ref[0, 0] = out_ref[0, 0] * (1.0 / (nsteps * 4 * a_ref.shape[0]))


def kernel(y_hat, y):
    n, num_class = y_hat.shape
    cvals = _sc_gather(y_hat.reshape(-1), y, num_class)
    c2 = cvals.reshape(n, 1)

    blk = 256
    grid = n // (4 * blk)

    def mk(q):
        return pl.BlockSpec((blk, num_class), lambda i, q=q: (4 * i + q, 0))

    def mkc(q):
        return pl.BlockSpec((blk, 1), lambda i, q=q: (4 * i + q, 0))

    out = pl.pallas_call(
        _body,
        grid=(grid,),
        in_specs=[mk(0), mk(1), mk(2), mk(3), mkc(0), mkc(1), mkc(2), mkc(3)],
        out_specs=pl.BlockSpec((1, 1), lambda i: (0, 0), memory_space=pltpu.SMEM),
        out_shape=jax.ShapeDtypeStruct((1, 1), jnp.float32),
    )(y_hat, y_hat, y_hat, y_hat, c2, c2, c2, c2)
    return out[0, 0]


# speedup vs baseline: 1.7671x; 1.7671x over previous
import jax
import jax.numpy as jnp
from jax.experimental import pallas as pl
from jax.experimental.pallas import tpu as pltpu

_SCALE = 30.0
_R2 = 0.7071067811865476
_LOG2E = 1.4426950408889634
_A = _SCALE * _LOG2E


def _psi(c):
    c = jnp.clip(c, -1.0, 1.0)
    c2 = c * c
    cos4 = 8.0 * c2 * c2 - 8.0 * c2 + 1.0
    k = (
        (c <= _R2).astype(jnp.int32)
        + (c <= 0.0).astype(jnp.int32)
        + (c <= -_R2).astype(jnp.int32)
    )
    co = jnp.where((k & 1) == 1, -1.0, 1.0)
    return co * cos4 - 2.0 * k.astype(jnp.float32)


def _sub_loss(yh, yv):
    cols = jax.lax.broadcasted_iota(jnp.int32, yh.shape, 1)
    mask = cols == yv
    c = jnp.sum(jnp.where(mask, yh, 0.0), axis=1, keepdims=True)
    psi = _psi(c)
    s0 = jnp.sum(jnp.exp2(yh * _A), axis=1, keepdims=True)
    s = s0 - jnp.exp2(c * _A) + jnp.exp2(psi * _A)
    lse = jnp.log(s)
    return jnp.sum(lse - _SCALE * psi)


def _body(a_ref, b_ref, c_ref, d_ref, ya_ref, yb_ref, yc_ref, yd_ref, out_ref):
    i = pl.program_id(0)
    nsteps = pl.num_programs(0)

    part = (
        _sub_loss(a_ref[...], ya_ref[...])
        + _sub_loss(b_ref[...], yb_ref[...])
        + _sub_loss(c_ref[...], yc_ref[...])
        + _sub_loss(d_ref[...], yd_ref[...])
    )

    @pl.when(i == 0)
    def _init():
        out_ref[0, 0] = 0.0

    out_ref[0, 0] += part

    @pl.when(i == nsteps - 1)
    def _final():
        out_ref[0, 0] = out_ref[0, 0] * (1.0 / (nsteps * 4 * a_ref.shape[0]))


def kernel(y_hat, y):
    n, num_class = y_hat.shape
    blk = 128
    grid = n // (4 * blk)
    y2 = y.reshape(n, 1)

    def mk(q):
        return pl.BlockSpec((blk, num_class), lambda i, q=q: (4 * i + q, 0))

    def mky(q):
        return pl.BlockSpec((blk, 1), lambda i, q=q: (4 * i + q, 0))

    out = pl.pallas_call(
        _body,
        grid=(grid,),
        in_specs=[mk(0), mk(1), mk(2), mk(3), mky(0), mky(1), mky(2), mky(3)],
        out_specs=pl.BlockSpec((1, 1), lambda i: (0, 0), memory_space=pltpu.SMEM),
        out_shape=jax.ShapeDtypeStruct((1, 1), jnp.float32),
    )(y_hat, y_hat, y_hat, y_hat, y2, y2, y2, y2)
    return out[0, 0]


# speedup vs baseline: 2.0432x; 1.1563x over previous
import jax
import jax.numpy as jnp
from jax.experimental import pallas as pl
from jax.experimental.pallas import tpu as pltpu

_SCALE = 30.0
_R2 = 0.7071067811865476
_LOG2E = 1.4426950408889634
_A = _SCALE * _LOG2E


def _psi(c):
    c = jnp.clip(c, -1.0, 1.0)
    c2 = c * c
    cos4 = 8.0 * c2 * c2 - 8.0 * c2 + 1.0
    k = (
        (c <= _R2).astype(jnp.int32)
        + (c <= 0.0).astype(jnp.int32)
        + (c <= -_R2).astype(jnp.int32)
    )
    co = jnp.where((k & 1) == 1, -1.0, 1.0)
    return co * cos4 - 2.0 * k.astype(jnp.float32)


def _sub_loss(yh, yv):
    cols = jax.lax.broadcasted_iota(jnp.int32, yh.shape, 1)
    mask = cols == yv
    c = jnp.sum(jnp.where(mask, yh, 0.0), axis=1, keepdims=True)
    psi = _psi(c)
    s0 = jnp.sum(jnp.exp2(yh * _A), axis=1, keepdims=True)
    s = s0 - jnp.exp2(c * _A) + jnp.exp2(psi * _A)
    lse = jnp.log(s)
    return jnp.sum(lse - _SCALE * psi)


def _body(a_ref, b_ref, c_ref, d_ref, ya_ref, yb_ref, yc_ref, yd_ref, out_ref):
    i = pl.program_id(0)
    nsteps = pl.num_programs(0)

    part = (
        _sub_loss(a_ref[...], ya_ref[...])
        + _sub_loss(b_ref[...], yb_ref[...])
        + _sub_loss(c_ref[...], yc_ref[...])
        + _sub_loss(d_ref[...], yd_ref[...])
    )

    @pl.when(i == 0)
    def _init():
        out_ref[0, 0] = 0.0

    out_ref[0, 0] += part

    @pl.when(i == nsteps - 1)
    def _final():
        out_ref[0, 0] = out_ref[0, 0] * (1.0 / (nsteps * 4 * a_ref.shape[0]))


def kernel(y_hat, y):
    n, num_class = y_hat.shape
    blk = 512
    grid = n // (4 * blk)
    y2 = y.reshape(n, 1)

    def mk(q):
        return pl.BlockSpec((blk, num_class), lambda i, q=q: (4 * i + q, 0))

    def mky(q):
        return pl.BlockSpec((blk, 1), lambda i, q=q: (4 * i + q, 0))

    out = pl.pallas_call(
        _body,
        grid=(grid,),
        in_specs=[mk(0), mk(1), mk(2), mk(3), mky(0), mky(1), mky(2), mky(3)],
        out_specs=pl.BlockSpec((1, 1), lambda i: (0, 0), memory_space=pltpu.SMEM),
        out_shape=jax.ShapeDtypeStruct((1, 1), jnp.float32),
    )(y_hat, y_hat, y_hat, y_hat, y2, y2, y2, y2)
    return out[0, 0]
